# baseline (speedup 1.0000x reference)
import functools

import jax
import jax.numpy as jnp
from jax import lax
from jax.experimental import pallas as pl
from jax.experimental.pallas import tpu as pltpu
from jax.experimental.pallas import tpu_sc as plsc

VOCAB = 37
HIDDEN = 128
NC, NS = 2, 16
NW = NC * NS
B = 4096 * 200
B_PER_W = B // NW
CHUNK = 128
N_CHUNKS = B_PER_W // CHUNK
NBUF = 6
PF = 4
IDX_ROWS_PER_W = B_PER_W // CHUNK

_mesh = plsc.VectorSubcoreMesh(core_axis_name="c", subcore_axis_name="s")


@functools.partial(
    pl.kernel,
    mesh=_mesh,
    out_type=jax.ShapeDtypeStruct((B, HIDDEN), jnp.float32),
    scratch_types=[
        pltpu.VMEM_SHARED((NS * VOCAB, HIDDEN), jnp.float32),
        pltpu.VMEM((IDX_ROWS_PER_W, CHUNK), jnp.int32),
        pltpu.VMEM((NBUF, CHUNK, HIDDEN), jnp.float32),
        pltpu.VMEM((1, HIDDEN), jnp.float32),
        pltpu.SemaphoreType.DMA,
        pltpu.SemaphoreType.DMA,
    ],
)
def _embed_gather(idx_hbm, table_hbm, out_hbm, table_sp, idx_v, rows_v, zrow_v,
                  gsem, wsem):
    cid = lax.axis_index("c")
    sid = lax.axis_index("s")
    wid = sid * NC + cid
    base = wid * B_PER_W

    pltpu.sync_copy(table_hbm, table_sp.at[pl.ds(sid * VOCAB, VOCAB)])
    for t in range(HIDDEN // 16):
        zrow_v[0, pl.ds(t * 16, 16)] = jnp.zeros((16,), jnp.float32)
    pltpu.sync_copy(zrow_v, table_sp.at[pl.ds(sid * VOCAB, 1)])

    pltpu.sync_copy(
        idx_hbm.at[pl.ds(wid * IDX_ROWS_PER_W, IDX_ROWS_PER_W)], idx_v)
    roff = sid * VOCAB

    def adj(r, _):
        for t in range(CHUNK // 16):
            idx_v[r, pl.ds(t * 16, 16)] = idx_v[r, pl.ds(t * 16, 16)] + roff
        return ()

    lax.fori_loop(0, IDX_ROWS_PER_W, adj, ())

    def fire_gather(c, p):
        pltpu.async_copy(table_sp.at[idx_v.at[c]], rows_v.at[p], gsem)

    def wait_gather(p):
        pltpu.make_async_copy(
            table_sp.at[idx_v.at[0]], rows_v.at[p], gsem).wait()

    def fire_write(c, p):
        pltpu.async_copy(
            rows_v.at[p], out_hbm.at[pl.ds(base + c * CHUNK, CHUNK)], wsem)

    def wait_write(p):
        pltpu.make_async_copy(
            rows_v.at[p], out_hbm.at[pl.ds(base, CHUNK)], wsem).wait()

    def step(c, p, wait_prev_write, prefetch):
        wait_gather(p)
        fire_write(c, p)
        if wait_prev_write:
            wait_write((p - 1) % NBUF)
        if prefetch:
            fire_gather(c + PF, (p + PF) % NBUF)

    for c in range(PF):
        fire_gather(c, c)
    step(0, 0, False, True)
    for c in range(1, PF):
        step(c, c, True, True)

    def body(g, _):
        for u in range(NBUF):
            c = PF + NBUF * g + u
            step(c, (PF + u) % NBUF, True, True)
        return ()

    lax.fori_loop(0, (N_CHUNKS - 2 * PF) // NBUF, body, ())

    for c in range(N_CHUNKS - PF, N_CHUNKS):
        step(c, c % NBUF, True, False)
    wait_write((N_CHUNKS - 1) % NBUF)


def kernel(indices, table):
    idx2d = indices.reshape(B // CHUNK, CHUNK)
    out = _embed_gather(idx2d, table)
    return out.reshape(4096, 200, HIDDEN)


# speedup vs baseline: 1.0152x; 1.0152x over previous
import functools

import jax
import jax.numpy as jnp
from jax import lax
from jax.experimental import pallas as pl
from jax.experimental.pallas import tpu as pltpu
from jax.experimental.pallas import tpu_sc as plsc

VOCAB = 37
HIDDEN = 128
NC, NS = 2, 16
NW = NC * NS
B = 4096 * 200
B_PER_W = B // NW
CHUNK = 128
N_CHUNKS = B_PER_W // CHUNK
NBUF = 6
PF = 4
IDX_ROWS_PER_W = B_PER_W // CHUNK

_mesh = plsc.VectorSubcoreMesh(core_axis_name="c", subcore_axis_name="s")


@functools.partial(
    pl.kernel,
    mesh=_mesh,
    out_type=jax.ShapeDtypeStruct((B, HIDDEN), jnp.float32),
    scratch_types=[
        pltpu.VMEM_SHARED((VOCAB, HIDDEN), jnp.float32),
        pltpu.VMEM((IDX_ROWS_PER_W, CHUNK), jnp.int32),
        pltpu.VMEM((NBUF, CHUNK, HIDDEN), jnp.float32),
        pltpu.VMEM((HIDDEN,), jnp.float32),
        pltpu.SemaphoreType.DMA,
        pltpu.SemaphoreType.DMA,
    ],
)
def _embed_gather(idx_hbm, table_hbm, out_hbm, table_sp, idx_v, rows_v, zrow_v,
                  gsem, wsem):
    cid = lax.axis_index("c")
    sid = lax.axis_index("s")
    wid = sid * NC + cid
    base = wid * B_PER_W

    @pl.when(sid == 0)
    def _():
        pltpu.sync_copy(table_hbm, table_sp)
        for t in range(HIDDEN // 16):
            zrow_v[pl.ds(t * 16, 16)] = jnp.zeros((16,), jnp.float32)
        pltpu.sync_copy(zrow_v, table_sp.at[0])

    pltpu.sync_copy(
        idx_hbm.at[pl.ds(wid * IDX_ROWS_PER_W, IDX_ROWS_PER_W)], idx_v)
    plsc.subcore_barrier()

    def fire_gather(c, p):
        pltpu.async_copy(table_sp.at[idx_v.at[c]], rows_v.at[p], gsem)

    def wait_gather(p):
        pltpu.make_async_copy(
            table_sp.at[idx_v.at[0]], rows_v.at[p], gsem).wait()

    def fire_write(c, p):
        pltpu.async_copy(
            rows_v.at[p], out_hbm.at[pl.ds(base + c * CHUNK, CHUNK)], wsem)

    def wait_write(p):
        pltpu.make_async_copy(
            rows_v.at[p], out_hbm.at[pl.ds(base, CHUNK)], wsem).wait()

    def step(c, p, wait_prev_write, prefetch):
        wait_gather(p)
        fire_write(c, p)
        if prefetch:
            fire_gather(c + PF, (p + PF) % NBUF)
        if wait_prev_write:
            wait_write((p - 1) % NBUF)

    for c in range(PF):
        fire_gather(c, c)
    step(0, 0, False, True)
    for c in range(1, PF):
        step(c, c, True, True)

    def body(g, _):
        for u in range(NBUF):
            c = PF + NBUF * g + u
            step(c, (PF + u) % NBUF, True, True)
        return ()

    lax.fori_loop(0, (N_CHUNKS - 2 * PF) // NBUF, body, ())

    for c in range(N_CHUNKS - PF, N_CHUNKS):
        step(c, c % NBUF, True, False)
    wait_write((N_CHUNKS - 1) % NBUF)


def kernel(indices, table):
    idx2d = indices.reshape(B // CHUNK, CHUNK)
    out = _embed_gather(idx2d, table)
    return out.reshape(4096, 200, HIDDEN)
